# Initial kernel scaffold; baseline (speedup 1.0000x reference)
import jax
import jax.numpy as jnp
from jax.experimental import pallas as pl


def kernel(x):
    raise NotImplementedError("write your pallas kernel here")


# speedup vs baseline: 15.7707x; 15.7707x over previous
import jax
import jax.numpy as jnp
from jax.experimental import pallas as pl
from jax.experimental.pallas import tpu as pltpu

_N_CHUNKS = 16
_BB = 128


def _pool_kernel(x_ref, o_ref):
    cw = x_ref.shape[1] // _N_CHUNKS
    c16 = [
        jnp.max(x_ref[:, i * cw : (i + 1) * cw], axis=-1, keepdims=True)
        for i in range(_N_CHUNKS)
    ]

    def pairmax(v):
        return [jnp.maximum(v[2 * i], v[2 * i + 1]) for i in range(len(v) // 2)]

    c8 = pairmax(c16)
    c4 = pairmax(c8)
    c2 = pairmax(c4)
    c1 = pairmax(c2)
    o_ref[...] = jnp.concatenate(c1 + c2 + c4 + c8 + c16, axis=-1)


def kernel(x):
    B, W = x.shape
    return pl.pallas_call(
        _pool_kernel,
        out_shape=jax.ShapeDtypeStruct((B, 31), x.dtype),
        grid=(B // _BB,),
        in_specs=[pl.BlockSpec((_BB, W), lambda i: (i, 0))],
        out_specs=pl.BlockSpec((_BB, 31), lambda i: (i, 0)),
        compiler_params=pltpu.CompilerParams(
            dimension_semantics=("arbitrary",),
        ),
    )(x)
